# baseline (speedup 1.0000x reference)
import jax
import jax.numpy as jnp
from jax import lax
from jax.experimental import pallas as pl
from jax.experimental.pallas import tpu as pltpu

DIM_ = 1024
NH_ = 16
HD_ = DIM_ // NH_
K_ = 16
BM_ = 256


def _layernorm(x, g, b):
    m = jnp.mean(x, axis=-1, keepdims=True)
    xc = x - m
    v = jnp.mean(xc * xc, axis=-1, keepdims=True)
    return xc * lax.rsqrt(v + 1e-5) * g + b


def _mmt(a, w):
    return lax.dot_general(a, w, (((1,), (1,)), ((), ())),
                           preferred_element_type=jnp.float32)


def _qkv_kernel(x_ref, wq_ref, wk_ref, wv_ref, bq_ref, bk_ref, bv_ref,
                g1_ref, beta1_ref, q_ref, k_ref, v_ref):
    xn = _layernorm(x_ref[...], g1_ref[...], beta1_ref[...]).astype(jnp.bfloat16)
    scale = 1.0 / jnp.sqrt(jnp.float32(HD_))
    q_ref[...] = ((_mmt(xn, wq_ref[...]) + bq_ref[...]) * scale).astype(jnp.bfloat16)
    k_ref[...] = (_mmt(xn, wk_ref[...]) + bk_ref[...]).astype(jnp.bfloat16)
    v_ref[...] = (_mmt(xn, wv_ref[...]) + bv_ref[...]).astype(jnp.bfloat16)


def _attn_kernel(c_ref, lt_ref, posq_ref, posall_ref, q_ref, k_ref, v_ref,
                 out_ref):
    c = c_ref[0]
    inv_tau = 1.0 / (jnp.exp(lt_ref[0]) + 1e-8)
    pq = posq_ref[0]
    pa = posall_ref[0]
    n = pa.shape[0]
    bm = pq.shape[0]
    qn = jnp.sum(pq * pq, axis=-1, keepdims=True)
    ones = jnp.ones((1, pq.shape[1]), jnp.float32)
    an = lax.dot_general(ones, pa * pa, (((1,), (1,)), ((), ())),
                         precision=lax.Precision.HIGHEST)
    qa = lax.dot_general(pq, pa, (((1,), (1,)), ((), ())),
                         precision=lax.Precision.HIGHEST)
    d2 = qn + an - 2.0 * qa
    num = 2.0 * c * d2
    den = (1.0 - c * qn) * (1.0 - c * an)
    arg = jnp.maximum(1.0 + num / (den + 1e-8), 1.0)
    dist = jnp.log(arg + jnp.sqrt(arg * arg - 1.0)) / jnp.sqrt(c)

    lane = lax.broadcasted_iota(jnp.int32, (bm, n), 1)
    rowg = (lax.broadcasted_iota(jnp.int32, (bm, n), 0)
            + pl.program_id(1) * bm)
    inf = jnp.float32(jnp.inf)
    work = jnp.where(lane == rowg, inf, dist)

    s_all = []
    for h in range(NH_):
        sl = pl.ds(h * HD_, HD_)
        s = lax.dot_general(q_ref[0, :, sl], k_ref[0, :, sl],
                            (((1,), (1,)), ((), ())),
                            preferred_element_type=jnp.float32)
        s_all.append(s.astype(jnp.bfloat16))

    for _ in range(K_ - 1):
        m = jnp.min(work, axis=-1, keepdims=True)
        work = jnp.where(work == m, inf, work)
    sel = work == inf
    g = jnp.where(sel, dist * (-inv_tau), -1e30).astype(jnp.bfloat16)

    for h in range(NH_):
        sl = pl.ds(h * HD_, HD_)
        p = jnp.exp(s_all[h] + g)
        denom = jnp.sum(p, axis=-1, keepdims=True).astype(jnp.float32)
        o = jnp.dot(p, v_ref[0, :, sl], preferred_element_type=jnp.float32)
        out_ref[0, :, sl] = (o / denom).astype(jnp.bfloat16)


def _ffn_kernel(attn_ref, x_ref, wo_ref, bo_ref, g2_ref, beta2_ref,
                w1_ref, b1_ref, w2_ref, b2_ref, out_ref):
    x1 = x_ref[...] + _mmt(attn_ref[...], wo_ref[...]) + bo_ref[...]
    t = _layernorm(x1, g2_ref[...], beta2_ref[...]).astype(jnp.bfloat16)
    h = _mmt(t, w1_ref[...]) + b1_ref[...]
    h = 0.5 * h * (1.0 + lax.erf(h * jnp.float32(0.7071067811865476)))
    out_ref[...] = x1 + _mmt(h.astype(jnp.bfloat16), w2_ref[...]) + b2_ref[...]


@jax.jit
def kernel(x, positions, c, Wq, bq, Wk, bk, Wv, bv, Wo, bo, W1, b1, W2, b2,
           g1, beta1, g2, beta2, log_tau):
    B, N, dim = x.shape
    BN = B * N
    P = positions.shape[-1]
    x2 = x.reshape(BN, dim)
    row = lambda a: a.reshape(1, -1)
    bf = lambda a: a.astype(jnp.bfloat16)

    q, k, v = pl.pallas_call(
        _qkv_kernel,
        grid=(BN // BM_,),
        in_specs=[
            pl.BlockSpec((BM_, dim), lambda i: (i, 0)),
            pl.BlockSpec((dim, dim), lambda i: (0, 0)),
            pl.BlockSpec((dim, dim), lambda i: (0, 0)),
            pl.BlockSpec((dim, dim), lambda i: (0, 0)),
            pl.BlockSpec((1, dim), lambda i: (0, 0)),
            pl.BlockSpec((1, dim), lambda i: (0, 0)),
            pl.BlockSpec((1, dim), lambda i: (0, 0)),
            pl.BlockSpec((1, dim), lambda i: (0, 0)),
            pl.BlockSpec((1, dim), lambda i: (0, 0)),
        ],
        out_specs=[
            pl.BlockSpec((BM_, dim), lambda i: (i, 0)),
            pl.BlockSpec((BM_, dim), lambda i: (i, 0)),
            pl.BlockSpec((BM_, dim), lambda i: (i, 0)),
        ],
        out_shape=[jax.ShapeDtypeStruct((BN, dim), jnp.bfloat16)] * 3,
    )(x2, bf(Wq), bf(Wk), bf(Wv), row(bq), row(bk), row(bv), row(g1), row(beta1))

    attn = pl.pallas_call(
        _attn_kernel,
        grid=(B, N // BM_),
        in_specs=[
            pl.BlockSpec(memory_space=pltpu.SMEM),
            pl.BlockSpec(memory_space=pltpu.SMEM),
            pl.BlockSpec((1, BM_, P), lambda b, i: (b, i, 0)),
            pl.BlockSpec((1, N, P), lambda b, i: (b, 0, 0)),
            pl.BlockSpec((1, BM_, dim), lambda b, i: (b, i, 0)),
            pl.BlockSpec((1, N, dim), lambda b, i: (b, 0, 0)),
            pl.BlockSpec((1, N, dim), lambda b, i: (b, 0, 0)),
        ],
        out_specs=pl.BlockSpec((1, BM_, dim), lambda b, i: (b, i, 0)),
        out_shape=jax.ShapeDtypeStruct((B, N, dim), jnp.bfloat16),
    )(c, log_tau.reshape(1), positions, positions, q.reshape(B, N, dim),
      k.reshape(B, N, dim), v.reshape(B, N, dim))

    out = pl.pallas_call(
        _ffn_kernel,
        grid=(BN // BM_,),
        in_specs=[
            pl.BlockSpec((BM_, dim), lambda i: (i, 0)),
            pl.BlockSpec((BM_, dim), lambda i: (i, 0)),
            pl.BlockSpec((dim, dim), lambda i: (0, 0)),
            pl.BlockSpec((1, dim), lambda i: (0, 0)),
            pl.BlockSpec((1, dim), lambda i: (0, 0)),
            pl.BlockSpec((1, dim), lambda i: (0, 0)),
            pl.BlockSpec((4 * dim, dim), lambda i: (0, 0)),
            pl.BlockSpec((1, 4 * dim), lambda i: (0, 0)),
            pl.BlockSpec((dim, 4 * dim), lambda i: (0, 0)),
            pl.BlockSpec((1, dim), lambda i: (0, 0)),
        ],
        out_specs=pl.BlockSpec((BM_, dim), lambda i: (i, 0)),
        out_shape=jax.ShapeDtypeStruct((BN, dim), jnp.float32),
    )(attn.reshape(BN, dim), x2, bf(Wo), row(bo), row(g2), row(beta2),
      bf(W1), row(b1), bf(W2), row(b2))

    return out.reshape(B, N, dim)


# speedup vs baseline: 1.0497x; 1.0497x over previous
import jax
import jax.numpy as jnp
from jax import lax
from jax.experimental import pallas as pl
from jax.experimental.pallas import tpu as pltpu

DIM_ = 1024
NH_ = 16
HD_ = DIM_ // NH_
K_ = 16
BM_ = 256


def _layernorm(x, g, b):
    m = jnp.mean(x, axis=-1, keepdims=True)
    xc = x - m
    v = jnp.mean(xc * xc, axis=-1, keepdims=True)
    return xc * lax.rsqrt(v + 1e-5) * g + b


def _mmt(a, w):
    return lax.dot_general(a, w, (((1,), (1,)), ((), ())),
                           preferred_element_type=jnp.float32)


def _qkv_kernel(x_ref, wq_ref, wk_ref, wv_ref, bq_ref, bk_ref, bv_ref,
                g1_ref, beta1_ref, q_ref, k_ref, v_ref):
    xn = _layernorm(x_ref[...], g1_ref[...], beta1_ref[...]).astype(jnp.bfloat16)
    scale = 1.0 / jnp.sqrt(jnp.float32(HD_))
    q_ref[...] = ((_mmt(xn, wq_ref[...]) + bq_ref[...]) * scale).astype(jnp.bfloat16)
    k_ref[...] = (_mmt(xn, wk_ref[...]) + bk_ref[...]).astype(jnp.bfloat16)
    v = _mmt(xn, wv_ref[...]) + bv_ref[...]
    bm = v.shape[0]
    io = lax.broadcasted_iota(jnp.int32, (bm, HD_), 1)
    onecol = (1 - jnp.minimum(io, 1)).astype(jnp.bfloat16)
    for h in range(NH_):
        v_ref[:, pl.ds(h * 2 * HD_, HD_)] = (
            v[:, h * HD_:(h + 1) * HD_].astype(jnp.bfloat16))
        v_ref[:, pl.ds(h * 2 * HD_ + HD_, HD_)] = onecol


def _attn_kernel(c_ref, lt_ref, posq_ref, posall_ref, q_ref, k_ref, v_ref,
                 out_ref):
    c = c_ref[0]
    inv_tau = 1.0 / (jnp.exp(lt_ref[0]) + 1e-8)
    pq = posq_ref[0]
    pa = posall_ref[0]
    n = pa.shape[0]
    bm = pq.shape[0]
    qn = jnp.sum(pq * pq, axis=-1, keepdims=True)
    ones = jnp.ones((1, pq.shape[1]), jnp.float32)
    an = lax.dot_general(ones, pa * pa, (((1,), (1,)), ((), ())),
                         precision=lax.Precision.HIGHEST)
    qa = lax.dot_general(pq, pa, (((1,), (1,)), ((), ())),
                         precision=lax.Precision.HIGHEST)
    d2 = qn + an - 2.0 * qa
    num = 2.0 * c * d2
    den = (1.0 - c * qn) * (1.0 - c * an)
    arg = jnp.maximum(1.0 + num / (den + 1e-8), 1.0)
    dist = jnp.log(arg + jnp.sqrt(arg * arg - 1.0)) / jnp.sqrt(c)

    lane = lax.broadcasted_iota(jnp.int32, (bm, n), 1)
    rowg = (lax.broadcasted_iota(jnp.int32, (bm, n), 0)
            + pl.program_id(1) * bm)
    inf = jnp.float32(jnp.inf)
    work = jnp.where(lane == rowg, inf, dist)
    for _ in range(K_ - 1):
        m = jnp.min(work, axis=-1, keepdims=True)
        work = jnp.where(work == m, inf, work)
    sel = work == inf
    g = jnp.where(sel, dist * (-inv_tau), -1e30).astype(jnp.bfloat16)

    for h in range(NH_):
        sl = pl.ds(h * HD_, HD_)
        s = lax.dot_general(q_ref[0, :, sl], k_ref[0, :, sl],
                            (((1,), (1,)), ((), ())),
                            preferred_element_type=jnp.float32)
        p = jnp.exp(s.astype(jnp.bfloat16) + g)
        ov = jnp.dot(p, v_ref[0, :, pl.ds(h * 2 * HD_, 2 * HD_)],
                     preferred_element_type=jnp.float32)
        out_ref[0, :, sl] = (ov[:, :HD_] / ov[:, HD_:HD_ + 1]).astype(jnp.bfloat16)


def _ffn_kernel(attn_ref, x_ref, wo_ref, bo_ref, g2_ref, beta2_ref,
                w1_ref, b1_ref, w2_ref, b2_ref, out_ref):
    x1 = x_ref[...] + _mmt(attn_ref[...], wo_ref[...]) + bo_ref[...]
    t = _layernorm(x1, g2_ref[...], beta2_ref[...]).astype(jnp.bfloat16)
    h = _mmt(t, w1_ref[...]) + b1_ref[...]
    h = 0.5 * h * (1.0 + lax.erf(h * jnp.float32(0.7071067811865476)))
    out_ref[...] = x1 + _mmt(h.astype(jnp.bfloat16), w2_ref[...]) + b2_ref[...]


@jax.jit
def kernel(x, positions, c, Wq, bq, Wk, bk, Wv, bv, Wo, bo, W1, b1, W2, b2,
           g1, beta1, g2, beta2, log_tau):
    B, N, dim = x.shape
    BN = B * N
    P = positions.shape[-1]
    x2 = x.reshape(BN, dim)
    row = lambda a: a.reshape(1, -1)
    bf = lambda a: a.astype(jnp.bfloat16)

    q, k, v = pl.pallas_call(
        _qkv_kernel,
        grid=(BN // BM_,),
        in_specs=[
            pl.BlockSpec((BM_, dim), lambda i: (i, 0)),
            pl.BlockSpec((dim, dim), lambda i: (0, 0)),
            pl.BlockSpec((dim, dim), lambda i: (0, 0)),
            pl.BlockSpec((dim, dim), lambda i: (0, 0)),
            pl.BlockSpec((1, dim), lambda i: (0, 0)),
            pl.BlockSpec((1, dim), lambda i: (0, 0)),
            pl.BlockSpec((1, dim), lambda i: (0, 0)),
            pl.BlockSpec((1, dim), lambda i: (0, 0)),
            pl.BlockSpec((1, dim), lambda i: (0, 0)),
        ],
        out_specs=[
            pl.BlockSpec((BM_, dim), lambda i: (i, 0)),
            pl.BlockSpec((BM_, dim), lambda i: (i, 0)),
            pl.BlockSpec((BM_, 2 * dim), lambda i: (i, 0)),
        ],
        out_shape=[jax.ShapeDtypeStruct((BN, dim), jnp.bfloat16)] * 2
        + [jax.ShapeDtypeStruct((BN, 2 * dim), jnp.bfloat16)],
    )(x2, bf(Wq), bf(Wk), bf(Wv), row(bq), row(bk), row(bv), row(g1), row(beta1))

    attn = pl.pallas_call(
        _attn_kernel,
        grid=(B, N // BM_),
        in_specs=[
            pl.BlockSpec(memory_space=pltpu.SMEM),
            pl.BlockSpec(memory_space=pltpu.SMEM),
            pl.BlockSpec((1, BM_, P), lambda b, i: (b, i, 0)),
            pl.BlockSpec((1, N, P), lambda b, i: (b, 0, 0)),
            pl.BlockSpec((1, BM_, dim), lambda b, i: (b, i, 0)),
            pl.BlockSpec((1, N, dim), lambda b, i: (b, 0, 0)),
            pl.BlockSpec((1, N, 2 * dim), lambda b, i: (b, 0, 0)),
        ],
        out_specs=pl.BlockSpec((1, BM_, dim), lambda b, i: (b, i, 0)),
        out_shape=jax.ShapeDtypeStruct((B, N, dim), jnp.bfloat16),
    )(c, log_tau.reshape(1), positions, positions, q.reshape(B, N, dim),
      k.reshape(B, N, dim), v.reshape(B, N, 2 * dim))

    out = pl.pallas_call(
        _ffn_kernel,
        grid=(BN // BM_,),
        in_specs=[
            pl.BlockSpec((BM_, dim), lambda i: (i, 0)),
            pl.BlockSpec((BM_, dim), lambda i: (i, 0)),
            pl.BlockSpec((dim, dim), lambda i: (0, 0)),
            pl.BlockSpec((1, dim), lambda i: (0, 0)),
            pl.BlockSpec((1, dim), lambda i: (0, 0)),
            pl.BlockSpec((1, dim), lambda i: (0, 0)),
            pl.BlockSpec((4 * dim, dim), lambda i: (0, 0)),
            pl.BlockSpec((1, 4 * dim), lambda i: (0, 0)),
            pl.BlockSpec((dim, 4 * dim), lambda i: (0, 0)),
            pl.BlockSpec((1, dim), lambda i: (0, 0)),
        ],
        out_specs=pl.BlockSpec((BM_, dim), lambda i: (i, 0)),
        out_shape=jax.ShapeDtypeStruct((BN, dim), jnp.float32),
    )(attn.reshape(BN, dim), x2, bf(Wo), row(bo), row(g2), row(beta2),
      bf(W1), row(b1), bf(W2), row(b2))

    return out.reshape(B, N, dim)


# speedup vs baseline: 1.1436x; 1.0895x over previous
import jax
import jax.numpy as jnp
from jax import lax
from jax.experimental import pallas as pl
from jax.experimental.pallas import tpu as pltpu

DIM_ = 1024
NH_ = 16
HD_ = DIM_ // NH_
K_ = 16
BM_ = 512


def _layernorm(x, g, b):
    m = jnp.mean(x, axis=-1, keepdims=True)
    xc = x - m
    v = jnp.mean(xc * xc, axis=-1, keepdims=True)
    return xc * lax.rsqrt(v + 1e-5) * g + b


def _mmt(a, w):
    return lax.dot_general(a, w, (((1,), (1,)), ((), ())),
                           preferred_element_type=jnp.float32)


def _qkv_kernel(x_ref, wq_ref, wk_ref, wv_ref, bq_ref, bk_ref, bv_ref,
                g1_ref, beta1_ref, q_ref, k_ref, v_ref):
    xn = _layernorm(x_ref[...], g1_ref[...], beta1_ref[...]).astype(jnp.bfloat16)
    scale = 1.0 / jnp.sqrt(jnp.float32(HD_))
    q_ref[...] = ((_mmt(xn, wq_ref[...]) + bq_ref[...]) * scale).astype(jnp.bfloat16)
    k_ref[...] = (_mmt(xn, wk_ref[...]) + bk_ref[...]).astype(jnp.bfloat16)
    v = _mmt(xn, wv_ref[...]) + bv_ref[...]
    bm = v.shape[0]
    io = lax.broadcasted_iota(jnp.int32, (bm, HD_), 1)
    onecol = (1 - jnp.minimum(io, 1)).astype(jnp.bfloat16)
    for h in range(NH_):
        v_ref[:, pl.ds(h * 2 * HD_, HD_)] = (
            v[:, h * HD_:(h + 1) * HD_].astype(jnp.bfloat16))
        v_ref[:, pl.ds(h * 2 * HD_ + HD_, HD_)] = onecol


def _attn_kernel(c_ref, lt_ref, posq_ref, posall_ref, q_ref, k_ref, v_ref,
                 out_ref):
    c = c_ref[0]
    inv_tau = 1.0 / (jnp.exp(lt_ref[0]) + 1e-8)
    pq = posq_ref[0]
    pa = posall_ref[0]
    n = pa.shape[0]
    bm = pq.shape[0]
    qn = jnp.sum(pq * pq, axis=-1, keepdims=True)
    ones = jnp.ones((1, pq.shape[1]), jnp.float32)
    an = lax.dot_general(ones, pa * pa, (((1,), (1,)), ((), ())),
                         precision=lax.Precision.HIGHEST)
    qa = lax.dot_general(pq, pa, (((1,), (1,)), ((), ())),
                         precision=lax.Precision.HIGHEST)
    d2 = qn + an - 2.0 * qa
    num = 2.0 * c * d2
    den = (1.0 - c * qn) * (1.0 - c * an)
    arg = jnp.maximum(1.0 + num / (den + 1e-8), 1.0)
    dist = jnp.log(arg + jnp.sqrt(arg * arg - 1.0)) / jnp.sqrt(c)

    lane = lax.broadcasted_iota(jnp.int32, (bm, n), 1)
    rowg = (lax.broadcasted_iota(jnp.int32, (bm, n), 0)
            + pl.program_id(1) * bm)
    inf = jnp.float32(jnp.inf)
    work = jnp.where(lane == rowg, inf, dist)
    for _ in range(K_ - 1):
        m = jnp.min(work, axis=-1, keepdims=True)
        work = jnp.where(work == m, inf, work)
    sel = work == inf
    g = jnp.where(sel, dist * (-inv_tau), -1e30).astype(jnp.bfloat16)

    for h in range(NH_):
        sl = pl.ds(h * HD_, HD_)
        s = lax.dot_general(q_ref[0, :, sl], k_ref[0, :, sl],
                            (((1,), (1,)), ((), ())),
                            preferred_element_type=jnp.float32)
        p = jnp.exp(s.astype(jnp.bfloat16) + g)
        ov = jnp.dot(p, v_ref[0, :, pl.ds(h * 2 * HD_, 2 * HD_)],
                     preferred_element_type=jnp.float32)
        out_ref[0, :, sl] = (ov[:, :HD_] / ov[:, HD_:HD_ + 1]).astype(jnp.bfloat16)


def _ffn_kernel(attn_ref, x_ref, wo_ref, bo_ref, g2_ref, beta2_ref,
                w1_ref, b1_ref, w2_ref, b2_ref, out_ref):
    x1 = x_ref[...] + _mmt(attn_ref[...], wo_ref[...]) + bo_ref[...]
    t = _layernorm(x1, g2_ref[...], beta2_ref[...]).astype(jnp.bfloat16)
    h = _mmt(t, w1_ref[...]) + b1_ref[...]
    h = 0.5 * h * (1.0 + lax.erf(h * jnp.float32(0.7071067811865476)))
    out_ref[...] = x1 + _mmt(h.astype(jnp.bfloat16), w2_ref[...]) + b2_ref[...]


@jax.jit
def kernel(x, positions, c, Wq, bq, Wk, bk, Wv, bv, Wo, bo, W1, b1, W2, b2,
           g1, beta1, g2, beta2, log_tau):
    B, N, dim = x.shape
    BN = B * N
    P = positions.shape[-1]
    x2 = x.reshape(BN, dim)
    row = lambda a: a.reshape(1, -1)
    bf = lambda a: a.astype(jnp.bfloat16)

    q, k, v = pl.pallas_call(
        _qkv_kernel,
        grid=(BN // BM_,),
        in_specs=[
            pl.BlockSpec((BM_, dim), lambda i: (i, 0)),
            pl.BlockSpec((dim, dim), lambda i: (0, 0)),
            pl.BlockSpec((dim, dim), lambda i: (0, 0)),
            pl.BlockSpec((dim, dim), lambda i: (0, 0)),
            pl.BlockSpec((1, dim), lambda i: (0, 0)),
            pl.BlockSpec((1, dim), lambda i: (0, 0)),
            pl.BlockSpec((1, dim), lambda i: (0, 0)),
            pl.BlockSpec((1, dim), lambda i: (0, 0)),
            pl.BlockSpec((1, dim), lambda i: (0, 0)),
        ],
        out_specs=[
            pl.BlockSpec((BM_, dim), lambda i: (i, 0)),
            pl.BlockSpec((BM_, dim), lambda i: (i, 0)),
            pl.BlockSpec((BM_, 2 * dim), lambda i: (i, 0)),
        ],
        out_shape=[jax.ShapeDtypeStruct((BN, dim), jnp.bfloat16)] * 2
        + [jax.ShapeDtypeStruct((BN, 2 * dim), jnp.bfloat16)],
    )(x2, bf(Wq), bf(Wk), bf(Wv), row(bq), row(bk), row(bv), row(g1), row(beta1))

    attn = pl.pallas_call(
        _attn_kernel,
        grid=(B, N // BM_),
        in_specs=[
            pl.BlockSpec(memory_space=pltpu.SMEM),
            pl.BlockSpec(memory_space=pltpu.SMEM),
            pl.BlockSpec((1, BM_, P), lambda b, i: (b, i, 0)),
            pl.BlockSpec((1, N, P), lambda b, i: (b, 0, 0)),
            pl.BlockSpec((1, BM_, dim), lambda b, i: (b, i, 0)),
            pl.BlockSpec((1, N, dim), lambda b, i: (b, 0, 0)),
            pl.BlockSpec((1, N, 2 * dim), lambda b, i: (b, 0, 0)),
        ],
        out_specs=pl.BlockSpec((1, BM_, dim), lambda b, i: (b, i, 0)),
        out_shape=jax.ShapeDtypeStruct((B, N, dim), jnp.bfloat16),
    )(c, log_tau.reshape(1), positions, positions, q.reshape(B, N, dim),
      k.reshape(B, N, dim), v.reshape(B, N, 2 * dim))

    out = pl.pallas_call(
        _ffn_kernel,
        grid=(BN // BM_,),
        in_specs=[
            pl.BlockSpec((BM_, dim), lambda i: (i, 0)),
            pl.BlockSpec((BM_, dim), lambda i: (i, 0)),
            pl.BlockSpec((dim, dim), lambda i: (0, 0)),
            pl.BlockSpec((1, dim), lambda i: (0, 0)),
            pl.BlockSpec((1, dim), lambda i: (0, 0)),
            pl.BlockSpec((1, dim), lambda i: (0, 0)),
            pl.BlockSpec((4 * dim, dim), lambda i: (0, 0)),
            pl.BlockSpec((1, 4 * dim), lambda i: (0, 0)),
            pl.BlockSpec((dim, 4 * dim), lambda i: (0, 0)),
            pl.BlockSpec((1, dim), lambda i: (0, 0)),
        ],
        out_specs=pl.BlockSpec((BM_, dim), lambda i: (i, 0)),
        out_shape=jax.ShapeDtypeStruct((BN, dim), jnp.float32),
    )(attn.reshape(BN, dim), x2, bf(Wo), row(bo), row(g2), row(beta2),
      bf(W1), row(b1), bf(W2), row(b2))

    return out.reshape(B, N, dim)
